# baseline (speedup 1.0000x reference)
import functools

import jax
import jax.numpy as jnp
from jax import lax
from jax.experimental import pallas as pl
from jax.experimental.pallas import tpu as pltpu
from jax.experimental.pallas import tpu_sc as plsc

N = 10000
E = 320000
D1 = 128
D2 = 64
NPAD = 10240
NC = 2
NS = 16
NW = NC * NS
LANES = 16

MP_B = 128
MP_NB = 81
MP_EPT = MP_NB * MP_B
E_EXT = NW * MP_EPT
RB = 64
RNB = MP_EPT // RB
RCH = 54
NSLOT = 4

_RSQRT_MAGIC = 0x5F3759DF


def _fast_rsqrt(d):
    i = lax.bitcast_convert_type(d, jnp.int32)
    i = jnp.full((LANES,), _RSQRT_MAGIC, jnp.int32) - lax.shift_right_logical(
        i, jnp.full((LANES,), 1, jnp.int32))
    y = lax.bitcast_convert_type(i, jnp.float32)
    half_d = 0.5 * d
    for _ in range(3):
        y = y * (1.5 - half_d * y * y)
    return y


_SC_PARAMS = pltpu.CompilerParams(needs_layout_passes=False,
                                  use_tc_tiling_on_sc=False)


def _norm_kernel_body(row_b, col_b, ew_b,
                      norm_out,
                      deg_sh, colbuf, ewbuf, rbuf,
                      dinvbuf, normstage, zbuf, dsems):
    c = lax.axis_index("c")
    s = lax.axis_index("s")
    wid = c * NS + s

    def _z(k, _):
        zbuf[pl.ds(k * LANES, LANES)] = jnp.zeros((LANES,), jnp.float32)
        return _
    lax.fori_loop(0, 640 // LANES, _z, None)
    pltpu.sync_copy(zbuf, deg_sh.at[pl.ds(s * 640, 640)])
    plsc.subcore_barrier()

    for t2 in range(2):
        pltpu.sync_copy(col_b.at[s * 2 + t2], colbuf)
        pltpu.sync_copy(ew_b.at[s * 2 + t2], ewbuf)

        def _deg(k, _):
            for u in range(2):
                j = 2 * k + u

                @pl.when(j >= 2)
                def _():
                    pltpu.make_async_copy(
                        ewbuf.at[0], deg_sh.at[colbuf.at[0]], dsems[u]).wait()

                pltpu.async_copy(ewbuf.at[j], deg_sh.at[colbuf.at[j]],
                                 dsems[u], add=True)
            return _
        lax.fori_loop(0, MP_NB // 2, _deg, None)
        pltpu.make_async_copy(ewbuf.at[0], deg_sh.at[colbuf.at[0]],
                              dsems[0]).wait()
        pltpu.async_copy(ewbuf.at[MP_NB - 1], deg_sh.at[colbuf.at[MP_NB - 1]],
                         dsems[0], add=True)
        pltpu.make_async_copy(ewbuf.at[0], deg_sh.at[colbuf.at[0]],
                              dsems[0]).wait()
        pltpu.make_async_copy(ewbuf.at[0], deg_sh.at[colbuf.at[0]],
                              dsems[1]).wait()
    plsc.subcore_barrier()

    pltpu.sync_copy(deg_sh, dinvbuf)

    def _rs(k, _):
        d = dinvbuf[pl.ds(k * LANES, LANES)]
        dinvbuf[pl.ds(k * LANES, LANES)] = _fast_rsqrt(d)
        return _
    lax.fori_loop(0, NPAD // LANES, _rs, None)

    pltpu.sync_copy(row_b.at[wid], rbuf)
    pltpu.sync_copy(col_b.at[wid], colbuf)
    pltpu.sync_copy(ew_b.at[wid], ewbuf)

    def _nrm(r, _):
        for q in range(MP_B // LANES):
            sl = pl.ds(q * LANES, LANES)
            dr = plsc.load_gather(dinvbuf, [rbuf[r, sl]])
            dc = plsc.load_gather(dinvbuf, [colbuf[r, sl]])
            normstage[r, sl] = dr * ewbuf[r, sl] * dc
        return _
    lax.fori_loop(0, MP_NB, _nrm, None)
    pltpu.sync_copy(normstage, norm_out.at[wid])


_norm_kernel = functools.partial(
    pl.kernel,
    out_type=jax.ShapeDtypeStruct((NW, MP_NB, MP_B), jnp.float32),
    mesh=plsc.VectorSubcoreMesh(core_axis_name="c", subcore_axis_name="s"),
    compiler_params=_SC_PARAMS,
    scratch_types=[
        pltpu.VMEM_SHARED((NPAD,), jnp.float32),
        pltpu.VMEM((MP_NB, MP_B), jnp.int32),
        pltpu.VMEM((MP_NB, MP_B), jnp.float32),
        pltpu.VMEM((MP_NB, MP_B), jnp.int32),
        pltpu.VMEM((NPAD,), jnp.float32),
        pltpu.VMEM((MP_NB, MP_B), jnp.float32),
        pltpu.VMEM((640,), jnp.float32),
        [pltpu.SemaphoreType.DMA for _ in range(2)],
    ],
)(_norm_kernel_body)


def _mp_body(d, h_hbm, row_hbm, col_hbm, norm_hbm, acc_out,
             row2d, col2d, norm2d, bslots, fslots, zbuf, acc_sh, gsems, ssems):
    c = lax.axis_index("c")
    s = lax.axis_index("s")
    wid = c * NS + s
    nh = d // 32

    def _z(r, _):
        def _zc(q, _):
            zbuf[r, pl.ds(q * LANES, LANES)] = jnp.zeros((LANES,), jnp.float32)
            return _
        lax.fori_loop(0, d // LANES, _zc, None)
        return _
    lax.fori_loop(0, 16, _z, None)

    def _zs(q, _):
        pltpu.sync_copy(zbuf, acc_sh.at[pl.ds(s * 640 + q * 16, 16), :])
        return _
    lax.fori_loop(0, 40, _zs, None)
    plsc.subcore_barrier()

    def _fire_g(j, p):
        pltpu.async_copy(h_hbm.at[row2d.at[j]], bslots[p], gsems[p])

    def _wait_g(j, p):
        pltpu.make_async_copy(h_hbm.at[row2d.at[j]], bslots[p], gsems[p]).wait()

    def _fire_s(j, p):
        pltpu.async_copy(fslots[p], acc_sh.at[col2d.at[j]], ssems[p], add=True)

    def _wait_s(p):
        pltpu.make_async_copy(fslots[p], acc_sh.at[col2d.at[0]],
                              ssems[p]).wait()

    iot2 = 2 * lax.iota(jnp.int32, LANES)

    def _scale(j, p):
        bbuf = bslots[p]
        fbuf = fslots[p]

        def _rows(k, _):
            for u in range(2):
                b = k * 2 + u
                sv = plsc.load_gather(norm2d.at[j],
                                      [jnp.full((LANES,), b, jnp.int32)])
                for q in range(nh):
                    x32 = bbuf[b, pl.ds(q * 32, 32)]
                    ev, od = plsc.unpack(x32, format=plsc.PackFormat.INTERLEAVED,
                                         preferred_element_type=jnp.float32)
                    idx = iot2 + (q * 32)
                    plsc.store_scatter(fbuf.at[b], [idx], ev * sv)
                    plsc.store_scatter(fbuf.at[b], [idx + 1], od * sv)
            return _
        lax.fori_loop(0, RB // 2, _rows, None)

    def _chunkstage(ch, _):
        pltpu.sync_copy(row_hbm.at[wid, pl.ds(ch * RCH, RCH), :], row2d)
        pltpu.sync_copy(col_hbm.at[wid, pl.ds(ch * RCH, RCH), :], col2d)
        pltpu.sync_copy(norm_hbm.at[wid, pl.ds(ch * RCH, RCH), :], norm2d)

        _fire_g(0, 0)
        _fire_g(1, 1)

        def _pair(k, _):
            for u in range(2):
                j = 2 * k + u

                @pl.when(j >= 2)
                def _():
                    _wait_s(u)

                _wait_g(j, u)
                _scale(j, u)
                _fire_s(j, u)

                @pl.when(j + 2 <= RCH - 1)
                def _():
                    _fire_g(j + 2, u)
            return _
        lax.fori_loop(0, RCH // 2, _pair, None)
        _wait_s(0)
        _wait_s(1)
        return _
    lax.fori_loop(0, RNB // RCH, _chunkstage, None)
    plsc.subcore_barrier()

    pltpu.sync_copy(acc_sh.at[pl.ds(s * 640, 640), :],
                    acc_out.at[c, pl.ds(s * 640, 640), :])


def _make_mp_kernel(d):
    return functools.partial(
        pl.kernel,
        out_type=jax.ShapeDtypeStruct((NC, NPAD, d), jnp.float32),
        mesh=plsc.VectorSubcoreMesh(core_axis_name="c", subcore_axis_name="s"),
        compiler_params=_SC_PARAMS,
        scratch_types=[
            pltpu.VMEM((RCH, RB), jnp.int32),
            pltpu.VMEM((RCH, RB), jnp.int32),
            pltpu.VMEM((RCH, RB), jnp.float32),
            [pltpu.VMEM((RB, d), jnp.bfloat16) for _ in range(2)],
            [pltpu.VMEM((RB, d), jnp.float32) for _ in range(2)],
            pltpu.VMEM((16, d), jnp.float32),
            pltpu.VMEM_SHARED((NPAD, d), jnp.float32),
            [pltpu.SemaphoreType.DMA for _ in range(2)],
            [pltpu.SemaphoreType.DMA for _ in range(2)],
        ],
    )(functools.partial(_mp_body, d))


_mp_kernel_128 = _make_mp_kernel(D1)
_mp_kernel_64 = _make_mp_kernel(D2)


def _mm_body(x_ref, w_ref, o_ref):
    o_ref[...] = jnp.dot(x_ref[...], w_ref[...],
                         preferred_element_type=jnp.float32).astype(jnp.bfloat16)


def _matmul1(x, w):
    grid = (N + 127) // 128
    return pl.pallas_call(
        _mm_body,
        out_shape=jax.ShapeDtypeStruct((N, D1), jnp.bfloat16),
        grid=(grid,),
        in_specs=[
            pl.BlockSpec((128, D1), lambda i: (i, 0)),
            pl.BlockSpec((D1, D1), lambda i: (0, 0)),
        ],
        out_specs=pl.BlockSpec((128, D1), lambda i: (i, 0)),
    )(x, w)


def _mid_body(acc_ref, b_ref, w_ref, o_ref):
    h = acc_ref[0] + acc_ref[1] + b_ref[...]
    h = jnp.maximum(h, 0.0)
    o_ref[...] = jnp.dot(h, w_ref[...],
                         preferred_element_type=jnp.float32).astype(jnp.bfloat16)


def _mid_layer(accp, b1, w2):
    return pl.pallas_call(
        _mid_body,
        out_shape=jax.ShapeDtypeStruct((NPAD, D2), jnp.bfloat16),
        grid=(NPAD // 128,),
        in_specs=[
            pl.BlockSpec((NC, 128, D1), lambda i: (0, i, 0)),
            pl.BlockSpec((1, D1), lambda i: (0, 0)),
            pl.BlockSpec((D1, D2), lambda i: (0, 0)),
        ],
        out_specs=pl.BlockSpec((128, D2), lambda i: (i, 0)),
    )(accp, b1, w2)


def _out_body(acc_ref, b_ref, o_ref):
    z = acc_ref[0] + acc_ref[1] + b_ref[...]
    m = jnp.max(z, axis=1, keepdims=True)
    ez = jnp.exp(z - m)
    lse = jnp.log(jnp.sum(ez, axis=1, keepdims=True))
    o_ref[...] = z - m - lse


def _out_layer(accp, b2):
    return pl.pallas_call(
        _out_body,
        out_shape=jax.ShapeDtypeStruct((NPAD, D2), jnp.float32),
        grid=(NPAD // 128,),
        in_specs=[
            pl.BlockSpec((NC, 128, D2), lambda i: (0, i, 0)),
            pl.BlockSpec((1, D2), lambda i: (0, 0)),
        ],
        out_specs=pl.BlockSpec((128, D2), lambda i: (i, 0)),
    )(accp, b2)


def kernel(x, adj_indices, adj_values, W1, b1, W2, b2):
    row = adj_indices[0].astype(jnp.int32)
    col = adj_indices[1].astype(jnp.int32)
    ew = adj_values

    loop = jnp.arange(N, dtype=jnp.int32)
    zpad_i = jnp.zeros((E_EXT - E - N,), jnp.int32)
    zpad_f = jnp.zeros((E_EXT - E - N,), jnp.float32)
    row_b = jnp.concatenate([row, loop, zpad_i]).reshape(NW, MP_NB, MP_B)
    col_b = jnp.concatenate([col, loop, zpad_i]).reshape(NW, MP_NB, MP_B)
    ew_b = jnp.concatenate([ew, jnp.ones((N,), jnp.float32),
                            zpad_f]).reshape(NW, MP_NB, MP_B)

    nrm_b = _norm_kernel(row_b, col_b, ew_b)

    row_r = row_b.reshape(NW, RNB, RB)
    col_r = col_b.reshape(NW, RNB, RB)
    nrm_r = nrm_b.reshape(NW, RNB, RB)

    h1 = _matmul1(x, W1)
    acc1 = _mp_kernel_128(h1, row_r, col_r, nrm_r)
    h2 = _mid_layer(acc1, b1.reshape(1, D1), W2)
    acc2 = _mp_kernel_64(h2, row_r, col_r, nrm_r)
    out = _out_layer(acc2, b2.reshape(1, D2))
    return out[:N]


# speedup vs baseline: 1.1173x; 1.1173x over previous
import functools

import jax
import jax.numpy as jnp
from jax import lax
from jax.experimental import pallas as pl
from jax.experimental.pallas import tpu as pltpu
from jax.experimental.pallas import tpu_sc as plsc

N = 10000
E = 320000
D1 = 128
D2 = 64
NPAD = 10240
NC = 2
NS = 16
NW = NC * NS
LANES = 16

MP_B = 128
MP_NB = 81
MP_EPT = MP_NB * MP_B
E_EXT = NW * MP_EPT
RB = 64
RNB = MP_EPT // RB
RCH = 54
NSLOT = 4

_RSQRT_MAGIC = 0x5F3759DF


def _fast_rsqrt(d):
    i = lax.bitcast_convert_type(d, jnp.int32)
    i = jnp.full((LANES,), _RSQRT_MAGIC, jnp.int32) - lax.shift_right_logical(
        i, jnp.full((LANES,), 1, jnp.int32))
    y = lax.bitcast_convert_type(i, jnp.float32)
    half_d = 0.5 * d
    for _ in range(3):
        y = y * (1.5 - half_d * y * y)
    return y


_SC_PARAMS = pltpu.CompilerParams(needs_layout_passes=False,
                                  use_tc_tiling_on_sc=False)


def _norm_kernel_body(row_b, col_b, ew_b,
                      norm_out,
                      deg_sh, colbuf, ewbuf, rbuf,
                      dinvbuf, normstage, zbuf, dsems):
    c = lax.axis_index("c")
    s = lax.axis_index("s")
    wid = c * NS + s

    def _z(k, _):
        zbuf[pl.ds(k * LANES, LANES)] = jnp.zeros((LANES,), jnp.float32)
        return _
    lax.fori_loop(0, 640 // LANES, _z, None)
    pltpu.sync_copy(zbuf, deg_sh.at[pl.ds(s * 640, 640)])
    plsc.subcore_barrier()

    for t2 in range(2):
        pltpu.sync_copy(col_b.at[s * 2 + t2], colbuf)
        pltpu.sync_copy(ew_b.at[s * 2 + t2], ewbuf)

        def _deg(k, _):
            for u in range(2):
                j = 2 * k + u

                @pl.when(j >= 2)
                def _():
                    pltpu.make_async_copy(
                        ewbuf.at[0], deg_sh.at[colbuf.at[0]], dsems[u]).wait()

                pltpu.async_copy(ewbuf.at[j], deg_sh.at[colbuf.at[j]],
                                 dsems[u], add=True)
            return _
        lax.fori_loop(0, MP_NB // 2, _deg, None)
        pltpu.make_async_copy(ewbuf.at[0], deg_sh.at[colbuf.at[0]],
                              dsems[0]).wait()
        pltpu.async_copy(ewbuf.at[MP_NB - 1], deg_sh.at[colbuf.at[MP_NB - 1]],
                         dsems[0], add=True)
        pltpu.make_async_copy(ewbuf.at[0], deg_sh.at[colbuf.at[0]],
                              dsems[0]).wait()
        pltpu.make_async_copy(ewbuf.at[0], deg_sh.at[colbuf.at[0]],
                              dsems[1]).wait()
    plsc.subcore_barrier()

    pltpu.sync_copy(deg_sh, dinvbuf)

    def _rs(k, _):
        d = dinvbuf[pl.ds(k * LANES, LANES)]
        dinvbuf[pl.ds(k * LANES, LANES)] = _fast_rsqrt(d)
        return _
    lax.fori_loop(0, NPAD // LANES, _rs, None)

    pltpu.sync_copy(row_b.at[wid], rbuf)
    pltpu.sync_copy(col_b.at[wid], colbuf)
    pltpu.sync_copy(ew_b.at[wid], ewbuf)

    def _nrm(r, _):
        for q in range(MP_B // LANES):
            sl = pl.ds(q * LANES, LANES)
            dr = plsc.load_gather(dinvbuf, [rbuf[r, sl]])
            dc = plsc.load_gather(dinvbuf, [colbuf[r, sl]])
            normstage[r, sl] = dr * ewbuf[r, sl] * dc
        return _
    lax.fori_loop(0, MP_NB, _nrm, None)
    pltpu.sync_copy(normstage, norm_out.at[wid])


_norm_kernel = functools.partial(
    pl.kernel,
    out_type=jax.ShapeDtypeStruct((NW, MP_NB, MP_B), jnp.float32),
    mesh=plsc.VectorSubcoreMesh(core_axis_name="c", subcore_axis_name="s"),
    compiler_params=_SC_PARAMS,
    scratch_types=[
        pltpu.VMEM_SHARED((NPAD,), jnp.float32),
        pltpu.VMEM((MP_NB, MP_B), jnp.int32),
        pltpu.VMEM((MP_NB, MP_B), jnp.float32),
        pltpu.VMEM((MP_NB, MP_B), jnp.int32),
        pltpu.VMEM((NPAD,), jnp.float32),
        pltpu.VMEM((MP_NB, MP_B), jnp.float32),
        pltpu.VMEM((640,), jnp.float32),
        [pltpu.SemaphoreType.DMA for _ in range(2)],
    ],
)(_norm_kernel_body)


def _mp_body(d, h_hbm, row_hbm, col_hbm, norm_hbm, acc_out,
             row2d, col2d, norm2d, slots, zbuf, acc_sh, gsems, ssems):
    c = lax.axis_index("c")
    s = lax.axis_index("s")
    wid = c * NS + s
    nq = d // LANES

    def _z(r, _):
        def _zc(q, _):
            zbuf[r, pl.ds(q * LANES, LANES)] = jnp.zeros((LANES,), jnp.float32)
            return _
        lax.fori_loop(0, nq, _zc, None)
        return _
    lax.fori_loop(0, 16, _z, None)

    def _zs(q, _):
        pltpu.sync_copy(zbuf, acc_sh.at[pl.ds(s * 640 + q * 16, 16), :])
        return _
    lax.fori_loop(0, 40, _zs, None)
    plsc.subcore_barrier()

    def _fire_g(j, p):
        pltpu.async_copy(h_hbm.at[row2d.at[j]], slots[p], gsems[p])

    def _wait_g(j, p):
        pltpu.make_async_copy(h_hbm.at[row2d.at[j]], slots[p], gsems[p]).wait()

    def _fire_s(j, p):
        pltpu.async_copy(slots[p], acc_sh.at[col2d.at[j]], ssems[p], add=True)

    def _wait_s(p):
        pltpu.make_async_copy(slots[p], acc_sh.at[col2d.at[0]], ssems[p]).wait()

    def _scale(j, p):
        buf = slots[p]

        def _rows(k, _):
            for u in range(4):
                b = k * 4 + u
                sv = plsc.load_gather(norm2d.at[j],
                                      [jnp.full((LANES,), b, jnp.int32)])
                for q in range(nq):
                    sl = pl.ds(q * LANES, LANES)
                    buf[b, sl] = buf[b, sl] * sv
            return _
        lax.fori_loop(0, RB // 4, _rows, None)

    def _chunkstage(ch, _):
        pltpu.sync_copy(row_hbm.at[wid, pl.ds(ch * RCH, RCH), :], row2d)
        pltpu.sync_copy(col_hbm.at[wid, pl.ds(ch * RCH, RCH), :], col2d)
        pltpu.sync_copy(norm_hbm.at[wid, pl.ds(ch * RCH, RCH), :], norm2d)

        _fire_g(0, 0)
        _fire_g(1, 1)

        def _quad(k, _):
            for u in range(NSLOT):
                j = k * NSLOT + u
                t = (u + 2) % NSLOT

                @pl.when(j >= 2)
                def _():
                    _wait_s(t)

                @pl.when(j + 2 <= RCH - 1)
                def _():
                    _fire_g(j + 2, t)

                _wait_g(j, u)
                _scale(j, u)
                _fire_s(j, u)
            return _
        lax.fori_loop(0, RCH // NSLOT, _quad, None)
        for u in range(2):
            j = (RCH // NSLOT) * NSLOT + u
            _wait_s((u + 2) % NSLOT)
            _wait_g(j, u)
            _scale(j, u)
            _fire_s(j, u)
        _wait_s(0)
        _wait_s(1)
        return _
    lax.fori_loop(0, RNB // RCH, _chunkstage, None)
    plsc.subcore_barrier()

    pltpu.sync_copy(acc_sh.at[pl.ds(s * 640, 640), :],
                    acc_out.at[c, pl.ds(s * 640, 640), :])


def _make_mp_kernel(d):
    return functools.partial(
        pl.kernel,
        out_type=jax.ShapeDtypeStruct((NC, NPAD, d), jnp.float32),
        mesh=plsc.VectorSubcoreMesh(core_axis_name="c", subcore_axis_name="s"),
        compiler_params=_SC_PARAMS,
        scratch_types=[
            pltpu.VMEM((RCH, RB), jnp.int32),
            pltpu.VMEM((RCH, RB), jnp.int32),
            pltpu.VMEM((RCH, RB), jnp.float32),
            [pltpu.VMEM((RB, d), jnp.float32) for _ in range(NSLOT)],
            pltpu.VMEM((16, d), jnp.float32),
            pltpu.VMEM_SHARED((NPAD, d), jnp.float32),
            [pltpu.SemaphoreType.DMA for _ in range(NSLOT)],
            [pltpu.SemaphoreType.DMA for _ in range(NSLOT)],
        ],
    )(functools.partial(_mp_body, d))


_mp_kernel_128 = _make_mp_kernel(D1)
_mp_kernel_64 = _make_mp_kernel(D2)


def _mm_body(x_ref, w_ref, o_ref):
    o_ref[...] = jnp.dot(x_ref[...], w_ref[...],
                         preferred_element_type=jnp.float32)


def _matmul1(x, w):
    grid = (N + 127) // 128
    return pl.pallas_call(
        _mm_body,
        out_shape=jax.ShapeDtypeStruct((N, D1), jnp.float32),
        grid=(grid,),
        in_specs=[
            pl.BlockSpec((128, D1), lambda i: (i, 0)),
            pl.BlockSpec((D1, D1), lambda i: (0, 0)),
        ],
        out_specs=pl.BlockSpec((128, D1), lambda i: (i, 0)),
    )(x, w)


def _mid_body(acc_ref, b_ref, w_ref, o_ref):
    h = acc_ref[0] + acc_ref[1] + b_ref[...]
    h = jnp.maximum(h, 0.0)
    o_ref[...] = jnp.dot(h, w_ref[...], preferred_element_type=jnp.float32)


def _mid_layer(accp, b1, w2):
    return pl.pallas_call(
        _mid_body,
        out_shape=jax.ShapeDtypeStruct((NPAD, D2), jnp.float32),
        grid=(NPAD // 128,),
        in_specs=[
            pl.BlockSpec((NC, 128, D1), lambda i: (0, i, 0)),
            pl.BlockSpec((1, D1), lambda i: (0, 0)),
            pl.BlockSpec((D1, D2), lambda i: (0, 0)),
        ],
        out_specs=pl.BlockSpec((128, D2), lambda i: (i, 0)),
    )(accp, b1, w2)


def _out_body(acc_ref, b_ref, o_ref):
    z = acc_ref[0] + acc_ref[1] + b_ref[...]
    m = jnp.max(z, axis=1, keepdims=True)
    ez = jnp.exp(z - m)
    lse = jnp.log(jnp.sum(ez, axis=1, keepdims=True))
    o_ref[...] = z - m - lse


def _out_layer(accp, b2):
    return pl.pallas_call(
        _out_body,
        out_shape=jax.ShapeDtypeStruct((NPAD, D2), jnp.float32),
        grid=(NPAD // 128,),
        in_specs=[
            pl.BlockSpec((NC, 128, D2), lambda i: (0, i, 0)),
            pl.BlockSpec((1, D2), lambda i: (0, 0)),
        ],
        out_specs=pl.BlockSpec((128, D2), lambda i: (i, 0)),
    )(accp, b2)


def kernel(x, adj_indices, adj_values, W1, b1, W2, b2):
    row = adj_indices[0].astype(jnp.int32)
    col = adj_indices[1].astype(jnp.int32)
    ew = adj_values

    loop = jnp.arange(N, dtype=jnp.int32)
    zpad_i = jnp.zeros((E_EXT - E - N,), jnp.int32)
    zpad_f = jnp.zeros((E_EXT - E - N,), jnp.float32)
    row_b = jnp.concatenate([row, loop, zpad_i]).reshape(NW, MP_NB, MP_B)
    col_b = jnp.concatenate([col, loop, zpad_i]).reshape(NW, MP_NB, MP_B)
    ew_b = jnp.concatenate([ew, jnp.ones((N,), jnp.float32),
                            zpad_f]).reshape(NW, MP_NB, MP_B)

    nrm_b = _norm_kernel(row_b, col_b, ew_b)

    row_r = row_b.reshape(NW, RNB, RB)
    col_r = col_b.reshape(NW, RNB, RB)
    nrm_r = nrm_b.reshape(NW, RNB, RB)

    h1 = _matmul1(x, W1)
    acc1 = _mp_kernel_128(h1, row_r, col_r, nrm_r)
    h2 = _mid_layer(acc1, b1.reshape(1, D1), W2)
    acc2 = _mp_kernel_64(h2, row_r, col_r, nrm_r)
    out = _out_layer(acc2, b2.reshape(1, D2))
    return out[:N]


# speedup vs baseline: 1.1999x; 1.0739x over previous
import functools

import jax
import jax.numpy as jnp
from jax import lax
from jax.experimental import pallas as pl
from jax.experimental.pallas import tpu as pltpu
from jax.experimental.pallas import tpu_sc as plsc

N = 10000
E = 320000
D1 = 128
D2 = 64
NPAD = 10240
NC = 2
NS = 16
NW = NC * NS
LANES = 16

MP_B = 128
MP_NB = 81
MP_EPT = MP_NB * MP_B
E_EXT = NW * MP_EPT
RB = 64
RNB = MP_EPT // RB
RCH = 54
NSLOT = 4

_RSQRT_MAGIC = 0x5F3759DF


def _fast_rsqrt(d):
    i = lax.bitcast_convert_type(d, jnp.int32)
    i = jnp.full((LANES,), _RSQRT_MAGIC, jnp.int32) - lax.shift_right_logical(
        i, jnp.full((LANES,), 1, jnp.int32))
    y = lax.bitcast_convert_type(i, jnp.float32)
    half_d = 0.5 * d
    for _ in range(3):
        y = y * (1.5 - half_d * y * y)
    return y


_SC_PARAMS = pltpu.CompilerParams(needs_layout_passes=False,
                                  use_tc_tiling_on_sc=False)


def _norm_kernel_body(row_b, col_b, ew_b,
                      norm_out,
                      deg_sh, colbuf, ewbuf, rbuf,
                      dinvbuf, normstage, zbuf, dsems):
    c = lax.axis_index("c")
    s = lax.axis_index("s")
    wid = c * NS + s

    def _z(k, _):
        zbuf[pl.ds(k * LANES, LANES)] = jnp.zeros((LANES,), jnp.float32)
        return _
    lax.fori_loop(0, 640 // LANES, _z, None)
    pltpu.sync_copy(zbuf, deg_sh.at[pl.ds(s * 640, 640)])
    plsc.subcore_barrier()

    for t2 in range(2):
        pltpu.sync_copy(col_b.at[s * 2 + t2], colbuf)
        pltpu.sync_copy(ew_b.at[s * 2 + t2], ewbuf)

        def _deg(k, _):
            for u in range(2):
                j = 2 * k + u

                @pl.when(j >= 2)
                def _():
                    pltpu.make_async_copy(
                        ewbuf.at[0], deg_sh.at[colbuf.at[0]], dsems[u]).wait()

                pltpu.async_copy(ewbuf.at[j], deg_sh.at[colbuf.at[j]],
                                 dsems[u], add=True)
            return _
        lax.fori_loop(0, MP_NB // 2, _deg, None)
        pltpu.make_async_copy(ewbuf.at[0], deg_sh.at[colbuf.at[0]],
                              dsems[0]).wait()
        pltpu.async_copy(ewbuf.at[MP_NB - 1], deg_sh.at[colbuf.at[MP_NB - 1]],
                         dsems[0], add=True)
        pltpu.make_async_copy(ewbuf.at[0], deg_sh.at[colbuf.at[0]],
                              dsems[0]).wait()
        pltpu.make_async_copy(ewbuf.at[0], deg_sh.at[colbuf.at[0]],
                              dsems[1]).wait()
    plsc.subcore_barrier()

    pltpu.sync_copy(deg_sh, dinvbuf)

    def _rs(k, _):
        d = dinvbuf[pl.ds(k * LANES, LANES)]
        dinvbuf[pl.ds(k * LANES, LANES)] = _fast_rsqrt(d)
        return _
    lax.fori_loop(0, NPAD // LANES, _rs, None)

    pltpu.sync_copy(row_b.at[wid], rbuf)
    pltpu.sync_copy(col_b.at[wid], colbuf)
    pltpu.sync_copy(ew_b.at[wid], ewbuf)

    def _nrm(r, _):
        for q in range(MP_B // LANES):
            sl = pl.ds(q * LANES, LANES)
            dr = plsc.load_gather(dinvbuf, [rbuf[r, sl]])
            dc = plsc.load_gather(dinvbuf, [colbuf[r, sl]])
            normstage[r, sl] = dr * ewbuf[r, sl] * dc
        return _
    lax.fori_loop(0, MP_NB, _nrm, None)
    pltpu.sync_copy(normstage, norm_out.at[wid])


_norm_kernel = functools.partial(
    pl.kernel,
    out_type=jax.ShapeDtypeStruct((NW, MP_NB, MP_B), jnp.float32),
    mesh=plsc.VectorSubcoreMesh(core_axis_name="c", subcore_axis_name="s"),
    compiler_params=_SC_PARAMS,
    scratch_types=[
        pltpu.VMEM_SHARED((NPAD,), jnp.float32),
        pltpu.VMEM((MP_NB, MP_B), jnp.int32),
        pltpu.VMEM((MP_NB, MP_B), jnp.float32),
        pltpu.VMEM((MP_NB, MP_B), jnp.int32),
        pltpu.VMEM((NPAD,), jnp.float32),
        pltpu.VMEM((MP_NB, MP_B), jnp.float32),
        pltpu.VMEM((640,), jnp.float32),
        [pltpu.SemaphoreType.DMA for _ in range(2)],
    ],
)(_norm_kernel_body)


def _mp_body(d, h_hbm, row_hbm, col_hbm, norm_hbm, acc_out,
             row2d, col2d, norm2d, slots, zbuf, acc_sh, gsems, ssems):
    c = lax.axis_index("c")
    s = lax.axis_index("s")
    wid = c * NS + s
    nq = d // LANES

    def _z(r, _):
        def _zc(q, _):
            zbuf[r, pl.ds(q * LANES, LANES)] = jnp.zeros((LANES,), jnp.float32)
            return _
        lax.fori_loop(0, nq, _zc, None)
        return _
    lax.fori_loop(0, 16, _z, None)

    def _zs(q, _):
        pltpu.sync_copy(zbuf, acc_sh.at[pl.ds(s * 640 + q * 16, 16), :])
        return _
    lax.fori_loop(0, 40, _zs, None)
    plsc.subcore_barrier()

    def _fire_g(j, p):
        pltpu.async_copy(h_hbm.at[row2d.at[j]], slots[p], gsems[p])

    def _wait_g(j, p):
        pltpu.make_async_copy(h_hbm.at[row2d.at[j]], slots[p], gsems[p]).wait()

    def _fire_s(j, p):
        pltpu.async_copy(slots[p], acc_sh.at[col2d.at[j]], ssems[p], add=True)

    def _wait_s(p):
        pltpu.make_async_copy(slots[p], acc_sh.at[col2d.at[0]], ssems[p]).wait()

    def _scale(j, p):
        buf = slots[p]

        def _rows(k, _):
            for u in range(4):
                b = k * 4 + u
                sv = plsc.load_gather(norm2d.at[j],
                                      [jnp.full((LANES,), b, jnp.int32)])
                for q in range(nq):
                    sl = pl.ds(q * LANES, LANES)
                    buf[b, sl] = buf[b, sl] * sv
            return _
        lax.fori_loop(0, RB // 4, _rows, None)

    def _chunkstage(ch, _):
        pltpu.sync_copy(row_hbm.at[wid, pl.ds(ch * RCH, RCH), :], row2d)
        pltpu.sync_copy(col_hbm.at[wid, pl.ds(ch * RCH, RCH), :], col2d)
        pltpu.sync_copy(norm_hbm.at[wid, pl.ds(ch * RCH, RCH), :], norm2d)

        _fire_g(0, 0)
        _fire_g(1, 1)

        def _quad(k, _):
            for u in range(NSLOT):
                j = k * NSLOT + u
                t = (u + 2) % NSLOT

                @pl.when(j >= 2)
                def _():
                    _wait_s(t)

                @pl.when(j + 2 <= RCH - 1)
                def _():
                    _fire_g(j + 2, t)

                _wait_g(j, u)
                _scale(j, u)
                _fire_s(j, u)
            return _
        lax.fori_loop(0, RCH // NSLOT, _quad, None)
        for u in range(2):
            j = (RCH // NSLOT) * NSLOT + u
            _wait_s((u + 2) % NSLOT)
            _wait_g(j, u)
            _scale(j, u)
            _fire_s(j, u)
        _wait_s(0)
        _wait_s(1)
        return _
    lax.fori_loop(0, RNB // RCH, _chunkstage, None)
    plsc.subcore_barrier()

    pltpu.sync_copy(acc_sh.at[pl.ds(s * 640, 640), :],
                    acc_out.at[c, pl.ds(s * 640, 640), :])


def _make_mp_kernel(d):
    return functools.partial(
        pl.kernel,
        out_type=jax.ShapeDtypeStruct((NC, NPAD, d), jnp.float32),
        mesh=plsc.VectorSubcoreMesh(core_axis_name="c", subcore_axis_name="s"),
        compiler_params=_SC_PARAMS,
        scratch_types=[
            pltpu.VMEM((RCH, RB), jnp.int32),
            pltpu.VMEM((RCH, RB), jnp.int32),
            pltpu.VMEM((RCH, RB), jnp.float32),
            [pltpu.VMEM((RB, d), jnp.float32) for _ in range(NSLOT)],
            pltpu.VMEM((16, d), jnp.float32),
            pltpu.VMEM_SHARED((NPAD, d), jnp.float32),
            [pltpu.SemaphoreType.DMA for _ in range(NSLOT)],
            [pltpu.SemaphoreType.DMA for _ in range(NSLOT)],
        ],
    )(functools.partial(_mp_body, d))


_mp_kernel_128 = _make_mp_kernel(D1)
_mp_kernel_64 = _make_mp_kernel(D2)


def _mm_body(x_ref, w_ref, o_ref):
    o_ref[...] = jnp.dot(x_ref[...], w_ref[...],
                         preferred_element_type=jnp.float32)


def _matmul1(x, w):
    grid = (N + 127) // 128
    return pl.pallas_call(
        _mm_body,
        out_shape=jax.ShapeDtypeStruct((N, D1), jnp.float32),
        grid=(grid,),
        in_specs=[
            pl.BlockSpec((128, D1), lambda i: (i, 0)),
            pl.BlockSpec((D1, D1), lambda i: (0, 0)),
        ],
        out_specs=pl.BlockSpec((128, D1), lambda i: (i, 0)),
    )(x, w)


def _mid_body(acc_ref, b_ref, w_ref, o_ref):
    h = acc_ref[0] + acc_ref[1] + b_ref[...]
    h = jnp.maximum(h, 0.0)
    o_ref[...] = jnp.dot(h, w_ref[...], preferred_element_type=jnp.float32)


def _mid_layer(accp, b1, w2):
    return pl.pallas_call(
        _mid_body,
        out_shape=jax.ShapeDtypeStruct((NPAD, D2), jnp.float32),
        grid=(NPAD // 128,),
        in_specs=[
            pl.BlockSpec((NC, 128, D1), lambda i: (0, i, 0)),
            pl.BlockSpec((1, D1), lambda i: (0, 0)),
            pl.BlockSpec((D1, D2), lambda i: (0, 0)),
        ],
        out_specs=pl.BlockSpec((128, D2), lambda i: (i, 0)),
    )(accp, b1, w2)


def _out_body(acc_ref, b_ref, o_ref):
    z = acc_ref[0] + acc_ref[1] + b_ref[...]
    m = jnp.max(z, axis=1, keepdims=True)
    ez = jnp.exp(z - m)
    lse = jnp.log(jnp.sum(ez, axis=1, keepdims=True))
    o_ref[...] = z - m - lse


def _out_layer(accp, b2):
    return pl.pallas_call(
        _out_body,
        out_shape=jax.ShapeDtypeStruct((N, D2), jnp.float32),
        grid=((N + 127) // 128,),
        in_specs=[
            pl.BlockSpec((NC, 128, D2), lambda i: (0, i, 0)),
            pl.BlockSpec((1, D2), lambda i: (0, 0)),
        ],
        out_specs=pl.BlockSpec((128, D2), lambda i: (i, 0)),
    )(accp, b2)


def kernel(x, adj_indices, adj_values, W1, b1, W2, b2):
    row = adj_indices[0].astype(jnp.int32)
    col = adj_indices[1].astype(jnp.int32)
    ew = adj_values

    loop = jnp.arange(N, dtype=jnp.int32)
    zpad_i = jnp.zeros((E_EXT - E - N,), jnp.int32)
    zpad_f = jnp.zeros((E_EXT - E - N,), jnp.float32)
    row_b = jnp.concatenate([row, loop, zpad_i]).reshape(NW, MP_NB, MP_B)
    col_b = jnp.concatenate([col, loop, zpad_i]).reshape(NW, MP_NB, MP_B)
    ew_b = jnp.concatenate([ew, jnp.ones((N,), jnp.float32),
                            zpad_f]).reshape(NW, MP_NB, MP_B)

    nrm_b = _norm_kernel(row_b, col_b, ew_b)

    row_r = row_b.reshape(NW, RNB, RB)
    col_r = col_b.reshape(NW, RNB, RB)
    nrm_r = nrm_b.reshape(NW, RNB, RB)

    h1 = _matmul1(x, W1)
    acc1 = _mp_kernel_128(h1, row_r, col_r, nrm_r)
    h2 = _mid_layer(acc1, b1.reshape(1, D1), W2)
    acc2 = _mp_kernel_64(h2, row_r, col_r, nrm_r)
    return _out_layer(acc2, b2.reshape(1, D2))
